# baseline (speedup 1.0000x reference)
import functools

import jax
import jax.numpy as jnp
from jax import lax
from jax.experimental import pallas as pl
from jax.experimental.pallas import tpu as pltpu
from jax.experimental.pallas import tpu_sc as plsc

BATCH = 4096
SEQ = 200
EMBED_DIM = 64
PAD_DIM = 128
VOCAB = 1000000

NUM_CORES = 2
NUM_SUBCORES = 16
NUM_WORKERS = NUM_CORES * NUM_SUBCORES
BLOCK_B = BATCH // NUM_WORKERS
NBUF = 4

_mesh = plsc.VectorSubcoreMesh(core_axis_name="c", subcore_axis_name="s")

NT_FULL = VOCAB // 128
NT_PER_W = NT_FULL // NUM_WORKERS
TAIL0 = NT_FULL * 128


@functools.partial(
    pl.kernel,
    mesh=_mesh,
    out_type=jax.ShapeDtypeStruct((VOCAB, PAD_DIM), jnp.float32),
    scratch_types=[
        pltpu.VMEM((2, EMBED_DIM, PAD_DIM), jnp.float32),
        pltpu.VMEM((2, PAD_DIM, PAD_DIM), jnp.float32),
        pltpu.VMEM((EMBED_DIM, EMBED_DIM), jnp.float32),
    ]
    + [pltpu.SemaphoreType.DMA] * 4,
    compiler_params=pltpu.CompilerParams(
        use_tc_tiling_on_sc=True, needs_layout_passes=False
    ),
)
def _prep(tv_hbm, tp_hbm, vbuf, obuf, vtail, *sems):
    sem_r = sems[:2]
    sem_w = sems[2:]
    wid = lax.axis_index("s") * NUM_CORES + lax.axis_index("c")
    iota16 = jnp.arange(16, dtype=jnp.int32)
    colpat = [(iota16 + j) % 16 for j in range(16)]

    def col_of(k):
        return (wid + NUM_WORKERS * k) * 128

    for p in range(2):
        pltpu.async_copy(
            tv_hbm.at[:, pl.ds(col_of(p), 128)], vbuf.at[p], sem_r[p]
        )

    def outer(g, carry):
        for p in range(2):
            k = g * 2 + p
            pltpu.make_async_copy(
                tv_hbm.at[:, pl.ds(0, 128)], vbuf.at[p], sem_r[p]
            ).wait()

            @pl.when(k >= 2)
            def _():
                pltpu.make_async_copy(
                    obuf.at[p], tp_hbm.at[pl.ds(0, 128)], sem_w[p]
                ).wait()

            @plsc.parallel_loop(0, 8, step=1, unroll=2)
            def _(tb):
                tcol = tb * 16 + iota16
                for db in range(4):
                    drow = db * 16 + iota16
                    for j in range(16):
                        tcolj = tb * 16 + colpat[j]
                        v = plsc.load_gather(vbuf.at[p], [drow, tcolj])
                        plsc.store_scatter(obuf.at[p], [tcolj, drow], v)

            pltpu.async_copy(
                obuf.at[p], tp_hbm.at[pl.ds(col_of(k), 128)], sem_w[p]
            )
            kn = k + 2

            @pl.when(kn < NT_PER_W)
            def _():
                pltpu.async_copy(
                    tv_hbm.at[:, pl.ds(col_of(kn), 128)], vbuf.at[p], sem_r[p]
                )

        return carry

    lax.fori_loop(0, NT_PER_W // 2, outer, 0)
    for p in range(2):
        pltpu.make_async_copy(
            obuf.at[p], tp_hbm.at[pl.ds(0, 128)], sem_w[p]
        ).wait()

    @pl.when(wid < 4)
    def _():
        c0 = (NT_FULL - 4 + wid) * 128
        pltpu.sync_copy(tv_hbm.at[:, pl.ds(c0, 128)], vbuf.at[0])

        @plsc.parallel_loop(0, 8, step=1, unroll=2)
        def _(tb):
            for db in range(4):
                drow = db * 16 + iota16
                for j in range(16):
                    tcolj = tb * 16 + colpat[j]
                    v = plsc.load_gather(vbuf.at[0], [drow, tcolj])
                    plsc.store_scatter(obuf.at[0], [tcolj, drow], v)

        pltpu.sync_copy(obuf.at[0], tp_hbm.at[pl.ds(c0, 128)])

    @pl.when(wid == 4)
    def _():
        pltpu.sync_copy(tv_hbm.at[:, pl.ds(TAIL0, 64)], vtail)

        @plsc.parallel_loop(0, 4, step=1, unroll=2)
        def _(tb):
            for db in range(4):
                drow = db * 16 + iota16
                for j in range(16):
                    tcolj = tb * 16 + colpat[j]
                    v = plsc.load_gather(vtail, [drow, tcolj])
                    plsc.store_scatter(obuf.at[1], [tcolj, drow], v)

        pltpu.sync_copy(
            obuf.at[1, pl.ds(0, 64)], tp_hbm.at[pl.ds(TAIL0, 64)]
        )


@functools.partial(
    pl.kernel,
    mesh=_mesh,
    out_type=jax.ShapeDtypeStruct((SEQ, EMBED_DIM, BATCH), jnp.float32),
    scratch_types=[
        pltpu.VMEM((SEQ, BLOCK_B), jnp.int32),
        pltpu.VMEM((NBUF, BLOCK_B, PAD_DIM), jnp.float32),
        pltpu.VMEM((2, EMBED_DIM, BLOCK_B), jnp.float32),
    ]
    + [pltpu.SemaphoreType.DMA] * (NBUF + 2),
    compiler_params=pltpu.CompilerParams(
        use_tc_tiling_on_sc=True, needs_layout_passes=False
    ),
)
def _embed_sc(xt_hbm, tp_hbm, out_hbm, idx_v, rows_v, tbuf, *sems):
    sem_g = sems[:NBUF]
    sem_w = sems[NBUF:]
    wid = lax.axis_index("s") * NUM_CORES + lax.axis_index("c")
    b0 = wid * BLOCK_B
    pltpu.sync_copy(xt_hbm.at[:, pl.ds(b0, BLOCK_B)], idx_v)

    for b in range(NBUF):
        pltpu.async_copy(tp_hbm.at[idx_v.at[b]], rows_v.at[b], sem_g[b])

    iota16 = jnp.arange(16, dtype=jnp.int32)
    colpat = [(iota16 + j) % 16 for j in range(16)]

    def outer(g, carry):
        for b in range(NBUF):
            s = g * NBUF + b
            tb = b % 2
            pltpu.make_async_copy(
                tp_hbm.at[idx_v.at[0]], rows_v.at[b], sem_g[b]
            ).wait()

            @pl.when(s >= 2)
            def _():
                pltpu.make_async_copy(
                    tbuf.at[tb], out_hbm.at[0, :, pl.ds(b0, BLOCK_B)], sem_w[tb]
                ).wait()

            @plsc.parallel_loop(0, 8, step=1, unroll=2)
            def _(kb):
                krow = kb * 16 + iota16
                for db in range(4):
                    for j in range(16):
                        dcol = db * 16 + colpat[j]
                        v = plsc.load_gather(rows_v.at[b], [krow, dcol])
                        plsc.store_scatter(tbuf.at[tb], [dcol, krow], v)

            pltpu.async_copy(
                tbuf.at[tb], out_hbm.at[s, :, pl.ds(b0, BLOCK_B)], sem_w[tb]
            )

            sn = s + NBUF

            @pl.when(sn < SEQ)
            def _():
                pltpu.async_copy(tp_hbm.at[idx_v.at[sn]], rows_v.at[b], sem_g[b])

        return carry

    lax.fori_loop(0, SEQ // NBUF, outer, 0)

    for tb in range(2):
        pltpu.make_async_copy(
            tbuf.at[tb], out_hbm.at[0, :, pl.ds(b0, BLOCK_B)], sem_w[tb]
        ).wait()


def kernel(x, table):
    tp = _prep(table.T)
    out5 = _embed_sc(x.T, tp)
    return out5.transpose(2, 0, 1)


# speedup vs baseline: 1.3640x; 1.3640x over previous
import functools

import jax
import jax.numpy as jnp
from jax import lax
from jax.experimental import pallas as pl
from jax.experimental.pallas import tpu as pltpu
from jax.experimental.pallas import tpu_sc as plsc

BATCH = 4096
SEQ = 200
EMBED_DIM = 64
PAD_DIM = 128
VOCAB = 1000000

NUM_CORES = 2
NUM_SUBCORES = 16
NUM_WORKERS = NUM_CORES * NUM_SUBCORES
BLOCK_B = BATCH // NUM_WORKERS
NBUF = 4

_mesh = plsc.VectorSubcoreMesh(core_axis_name="c", subcore_axis_name="s")

NT_FULL = VOCAB // 128
NT_PER_W = NT_FULL // NUM_WORKERS
TAIL0 = NT_FULL * 128


@functools.partial(
    pl.kernel,
    mesh=_mesh,
    out_type=jax.ShapeDtypeStruct((VOCAB, PAD_DIM), jnp.float32),
    scratch_types=[
        pltpu.VMEM((2, EMBED_DIM, PAD_DIM), jnp.float32),
        pltpu.VMEM((2, PAD_DIM, PAD_DIM), jnp.float32),
        pltpu.VMEM((EMBED_DIM, EMBED_DIM), jnp.float32),
    ]
    + [pltpu.SemaphoreType.DMA] * 4,
    compiler_params=pltpu.CompilerParams(
        use_tc_tiling_on_sc=True, needs_layout_passes=False
    ),
)
def _prep(tv_hbm, tp_hbm, vbuf, obuf, vtail, *sems):
    sem_r = sems[:2]
    sem_w = sems[2:]
    wid = lax.axis_index("s") * NUM_CORES + lax.axis_index("c")
    iota16 = jnp.arange(16, dtype=jnp.int32)
    colpat = [(iota16 + j) % 16 for j in range(16)]

    def col_of(k):
        return (wid + NUM_WORKERS * k) * 128

    for p in range(2):
        pltpu.async_copy(
            tv_hbm.at[:, pl.ds(col_of(p), 128)], vbuf.at[p], sem_r[p]
        )

    def outer(g, carry):
        for p in range(2):
            k = g * 2 + p
            pltpu.make_async_copy(
                tv_hbm.at[:, pl.ds(0, 128)], vbuf.at[p], sem_r[p]
            ).wait()

            @pl.when(k >= 2)
            def _():
                pltpu.make_async_copy(
                    obuf.at[p], tp_hbm.at[pl.ds(0, 128)], sem_w[p]
                ).wait()

            @plsc.parallel_loop(0, 8, step=1, unroll=2)
            def _(tb):
                tcol = tb * 16 + iota16
                for db in range(4):
                    drow = db * 16 + iota16
                    for j in range(16):
                        tcolj = tb * 16 + colpat[j]
                        v = plsc.load_gather(vbuf.at[p], [drow, tcolj])
                        plsc.store_scatter(obuf.at[p], [tcolj, drow], v)

            pltpu.async_copy(
                obuf.at[p], tp_hbm.at[pl.ds(col_of(k), 128)], sem_w[p]
            )
            kn = k + 2

            @pl.when(kn < NT_PER_W)
            def _():
                pltpu.async_copy(
                    tv_hbm.at[:, pl.ds(col_of(kn), 128)], vbuf.at[p], sem_r[p]
                )

        return carry

    lax.fori_loop(0, NT_PER_W // 2, outer, 0)
    for p in range(2):
        pltpu.make_async_copy(
            obuf.at[p], tp_hbm.at[pl.ds(0, 128)], sem_w[p]
        ).wait()

    @pl.when(wid < 4)
    def _():
        c0 = (NT_FULL - 4 + wid) * 128
        pltpu.sync_copy(tv_hbm.at[:, pl.ds(c0, 128)], vbuf.at[0])

        @plsc.parallel_loop(0, 8, step=1, unroll=2)
        def _(tb):
            for db in range(4):
                drow = db * 16 + iota16
                for j in range(16):
                    tcolj = tb * 16 + colpat[j]
                    v = plsc.load_gather(vbuf.at[0], [drow, tcolj])
                    plsc.store_scatter(obuf.at[0], [tcolj, drow], v)

        pltpu.sync_copy(obuf.at[0], tp_hbm.at[pl.ds(c0, 128)])

    @pl.when(wid == 4)
    def _():
        pltpu.sync_copy(tv_hbm.at[:, pl.ds(TAIL0, 64)], vtail)

        @plsc.parallel_loop(0, 4, step=1, unroll=2)
        def _(tb):
            for db in range(4):
                drow = db * 16 + iota16
                for j in range(16):
                    tcolj = tb * 16 + colpat[j]
                    v = plsc.load_gather(vtail, [drow, tcolj])
                    plsc.store_scatter(obuf.at[1], [tcolj, drow], v)

        pltpu.sync_copy(
            obuf.at[1, pl.ds(0, 64)], tp_hbm.at[pl.ds(TAIL0, 64)]
        )


@functools.partial(
    pl.kernel,
    mesh=_mesh,
    out_type=jax.ShapeDtypeStruct((SEQ, EMBED_DIM, BATCH), jnp.float32),
    scratch_types=[
        pltpu.VMEM((SEQ, BLOCK_B), jnp.int32),
        pltpu.VMEM((NBUF, BLOCK_B, PAD_DIM), jnp.float32),
        pltpu.VMEM((2, EMBED_DIM, BLOCK_B), jnp.float32),
    ]
    + [pltpu.SemaphoreType.DMA] * (NBUF + 2),
    compiler_params=pltpu.CompilerParams(
        use_tc_tiling_on_sc=True, needs_layout_passes=False
    ),
)
def _embed_sc(xt_hbm, tp_hbm, out_hbm, idx_v, rows_v, tbuf, *sems):
    sem_g = sems[:NBUF]
    sem_w = sems[NBUF:]
    wid = lax.axis_index("s") * NUM_CORES + lax.axis_index("c")
    b0 = wid * BLOCK_B
    pltpu.sync_copy(xt_hbm.at[:, pl.ds(b0, BLOCK_B)], idx_v)

    for b in range(NBUF):
        pltpu.async_copy(tp_hbm.at[idx_v.at[b]], rows_v.at[b], sem_g[b])

    iota16 = jnp.arange(16, dtype=jnp.int32)
    colpat = [(iota16 + j) % 16 for j in range(16)]

    def outer(g, carry):
        for b in range(NBUF):
            s = g * NBUF + b
            tb = b % 2
            pltpu.make_async_copy(
                tp_hbm.at[idx_v.at[0]], rows_v.at[b], sem_g[b]
            ).wait()

            @pl.when(s >= 2)
            def _():
                pltpu.make_async_copy(
                    tbuf.at[tb], out_hbm.at[0, :, pl.ds(b0, BLOCK_B)], sem_w[tb]
                ).wait()

            @plsc.parallel_loop(0, 8, step=1, unroll=4)
            def _(kb):
                krow = kb * 16 + iota16
                for db in range(4):
                    for j in range(16):
                        dcol = db * 16 + colpat[j]
                        v = plsc.load_gather(rows_v.at[b], [krow, dcol])
                        plsc.store_scatter(tbuf.at[tb], [dcol, krow], v)

            pltpu.async_copy(
                tbuf.at[tb], out_hbm.at[s, :, pl.ds(b0, BLOCK_B)], sem_w[tb]
            )

            sn = s + NBUF

            @pl.when(sn < SEQ)
            def _():
                pltpu.async_copy(tp_hbm.at[idx_v.at[sn]], rows_v.at[b], sem_g[b])

        return carry

    lax.fori_loop(0, SEQ // NBUF, outer, 0)

    for tb in range(2):
        pltpu.make_async_copy(
            tbuf.at[tb], out_hbm.at[0, :, pl.ds(b0, BLOCK_B)], sem_w[tb]
        ).wait()


def kernel(x, table):
    tp = jnp.pad(table, ((0, 0), (0, PAD_DIM - EMBED_DIM)))
    out5 = _embed_sc(x.T, tp)
    return out5.transpose(2, 0, 1)


# speedup vs baseline: 1.3654x; 1.0010x over previous
import functools

import jax
import jax.numpy as jnp
from jax import lax
from jax.experimental import pallas as pl
from jax.experimental.pallas import tpu as pltpu
from jax.experimental.pallas import tpu_sc as plsc

BATCH = 4096
SEQ = 200
EMBED_DIM = 64
PAD_DIM = 128
VOCAB = 1000000

NUM_CORES = 2
NUM_SUBCORES = 16
NUM_WORKERS = NUM_CORES * NUM_SUBCORES
BLOCK_B = BATCH // NUM_WORKERS
NBUF = 4

_mesh = plsc.VectorSubcoreMesh(core_axis_name="c", subcore_axis_name="s")

NT_FULL = VOCAB // 128
NT_PER_W = NT_FULL // NUM_WORKERS
TAIL0 = NT_FULL * 128


@functools.partial(
    pl.kernel,
    mesh=_mesh,
    out_type=jax.ShapeDtypeStruct((VOCAB, PAD_DIM), jnp.float32),
    scratch_types=[
        pltpu.VMEM((2, EMBED_DIM, PAD_DIM), jnp.float32),
        pltpu.VMEM((2, PAD_DIM, PAD_DIM), jnp.float32),
        pltpu.VMEM((EMBED_DIM, EMBED_DIM), jnp.float32),
    ]
    + [pltpu.SemaphoreType.DMA] * 4,
    compiler_params=pltpu.CompilerParams(
        use_tc_tiling_on_sc=True, needs_layout_passes=False
    ),
)
def _prep(tv_hbm, tp_hbm, vbuf, obuf, vtail, *sems):
    sem_r = sems[:2]
    sem_w = sems[2:]
    wid = lax.axis_index("s") * NUM_CORES + lax.axis_index("c")
    iota16 = jnp.arange(16, dtype=jnp.int32)
    colpat = [(iota16 + j) % 16 for j in range(16)]

    def col_of(k):
        return (wid + NUM_WORKERS * k) * 128

    for p in range(2):
        pltpu.async_copy(
            tv_hbm.at[:, pl.ds(col_of(p), 128)], vbuf.at[p], sem_r[p]
        )

    def outer(g, carry):
        for p in range(2):
            k = g * 2 + p
            pltpu.make_async_copy(
                tv_hbm.at[:, pl.ds(0, 128)], vbuf.at[p], sem_r[p]
            ).wait()

            @pl.when(k >= 2)
            def _():
                pltpu.make_async_copy(
                    obuf.at[p], tp_hbm.at[pl.ds(0, 128)], sem_w[p]
                ).wait()

            @plsc.parallel_loop(0, 8, step=1, unroll=2)
            def _(tb):
                tcol = tb * 16 + iota16
                for db in range(4):
                    drow = db * 16 + iota16
                    for j in range(16):
                        tcolj = tb * 16 + colpat[j]
                        v = plsc.load_gather(vbuf.at[p], [drow, tcolj])
                        plsc.store_scatter(obuf.at[p], [tcolj, drow], v)

            pltpu.async_copy(
                obuf.at[p], tp_hbm.at[pl.ds(col_of(k), 128)], sem_w[p]
            )
            kn = k + 2

            @pl.when(kn < NT_PER_W)
            def _():
                pltpu.async_copy(
                    tv_hbm.at[:, pl.ds(col_of(kn), 128)], vbuf.at[p], sem_r[p]
                )

        return carry

    lax.fori_loop(0, NT_PER_W // 2, outer, 0)
    for p in range(2):
        pltpu.make_async_copy(
            obuf.at[p], tp_hbm.at[pl.ds(0, 128)], sem_w[p]
        ).wait()

    @pl.when(wid < 4)
    def _():
        c0 = (NT_FULL - 4 + wid) * 128
        pltpu.sync_copy(tv_hbm.at[:, pl.ds(c0, 128)], vbuf.at[0])

        @plsc.parallel_loop(0, 8, step=1, unroll=2)
        def _(tb):
            for db in range(4):
                drow = db * 16 + iota16
                for j in range(16):
                    tcolj = tb * 16 + colpat[j]
                    v = plsc.load_gather(vbuf.at[0], [drow, tcolj])
                    plsc.store_scatter(obuf.at[0], [tcolj, drow], v)

        pltpu.sync_copy(obuf.at[0], tp_hbm.at[pl.ds(c0, 128)])

    @pl.when(wid == 4)
    def _():
        pltpu.sync_copy(tv_hbm.at[:, pl.ds(TAIL0, 64)], vtail)

        @plsc.parallel_loop(0, 4, step=1, unroll=2)
        def _(tb):
            for db in range(4):
                drow = db * 16 + iota16
                for j in range(16):
                    tcolj = tb * 16 + colpat[j]
                    v = plsc.load_gather(vtail, [drow, tcolj])
                    plsc.store_scatter(obuf.at[1], [tcolj, drow], v)

        pltpu.sync_copy(
            obuf.at[1, pl.ds(0, 64)], tp_hbm.at[pl.ds(TAIL0, 64)]
        )


@functools.partial(
    pl.kernel,
    mesh=_mesh,
    out_type=jax.ShapeDtypeStruct((SEQ, EMBED_DIM, BATCH), jnp.float32),
    scratch_types=[
        pltpu.VMEM((SEQ, BLOCK_B), jnp.int32),
        pltpu.VMEM((NBUF, BLOCK_B, PAD_DIM), jnp.float32),
        pltpu.VMEM((2, EMBED_DIM, BLOCK_B), jnp.float32),
    ]
    + [pltpu.SemaphoreType.DMA] * (NBUF + 2),
    compiler_params=pltpu.CompilerParams(
        use_tc_tiling_on_sc=True, needs_layout_passes=False
    ),
)
def _embed_sc(xt_hbm, tp_hbm, out_hbm, idx_v, rows_v, tbuf, *sems):
    sem_g = sems[:NBUF]
    sem_w = sems[NBUF:]
    wid = lax.axis_index("s") * NUM_CORES + lax.axis_index("c")
    b0 = wid * BLOCK_B
    pltpu.sync_copy(xt_hbm.at[:, pl.ds(b0, BLOCK_B)], idx_v)

    for b in range(NBUF):
        pltpu.async_copy(tp_hbm.at[idx_v.at[b]], rows_v.at[b], sem_g[b])

    iota16 = jnp.arange(16, dtype=jnp.int32)
    colpat = [(iota16 + j) % 16 for j in range(16)]

    def outer(g, carry):
        for b in range(NBUF):
            s = g * NBUF + b
            tb = b % 2
            pltpu.make_async_copy(
                tp_hbm.at[idx_v.at[0]], rows_v.at[b], sem_g[b]
            ).wait()

            sn = s + NBUF - 1
            bp = (b + NBUF - 1) % NBUF

            @pl.when(jnp.logical_and(s >= 1, sn < SEQ))
            def _():
                pltpu.async_copy(tp_hbm.at[idx_v.at[sn]], rows_v.at[bp], sem_g[bp])

            @pl.when(s >= 2)
            def _():
                pltpu.make_async_copy(
                    tbuf.at[tb], out_hbm.at[0, :, pl.ds(b0, BLOCK_B)], sem_w[tb]
                ).wait()

            @plsc.parallel_loop(0, 8, step=1, unroll=4)
            def _(kb):
                krow = kb * 16 + iota16
                for db in range(4):
                    for j in range(16):
                        dcol = db * 16 + colpat[j]
                        v = plsc.load_gather(rows_v.at[b], [krow, dcol])
                        plsc.store_scatter(tbuf.at[tb], [dcol, krow], v)

            pltpu.async_copy(
                tbuf.at[tb], out_hbm.at[s, :, pl.ds(b0, BLOCK_B)], sem_w[tb]
            )

        return carry

    lax.fori_loop(0, SEQ // NBUF, outer, 0)

    for tb in range(2):
        pltpu.make_async_copy(
            tbuf.at[tb], out_hbm.at[0, :, pl.ds(b0, BLOCK_B)], sem_w[tb]
        ).wait()


def kernel(x, table):
    tp = jnp.pad(table, ((0, 0), (0, PAD_DIM - EMBED_DIM)))
    out5 = _embed_sc(x.T, tp)
    return out5.transpose(2, 0, 1)
